# baseline (speedup 1.0000x reference)
import functools

import jax
import jax.numpy as jnp
from jax import lax
from jax.experimental import pallas as pl
from jax.experimental.pallas import tpu as pltpu
from jax.experimental.pallas import tpu_sc as plsc

N = 4096
E = 262144
D = 128
T = 2
TAU = 0.1
EPS = 1e-5
BE = 4096
NB = E // BE

_F32 = jnp.float32


def _dot(a, b):
    return jnp.dot(a, b, preferred_element_type=_F32)


def _bn_full(x, gamma, beta):
    m = jnp.mean(x, 0, keepdims=True)
    v = jnp.mean((x - m) * (x - m), 0, keepdims=True)
    return (x - m) / jnp.sqrt(v + EPS) * gamma + beta


def _scale_shift(stats_ref, gamma, beta):
    mean = stats_ref[0:1, :] * (1.0 / E)
    ex2 = stats_ref[1:2, :] * (1.0 / E)
    var = ex2 - mean * mean
    scale = gamma / jnp.sqrt(var + EPS)
    shift = beta - mean * scale
    return scale, shift


def _node_pre_body(d, w11, b11, w12, b12, g1, be1, x1_o):
    x = jnp.maximum(_dot(d[...], w11[...]) + b11[...], 0.0)
    x = jnp.maximum(_dot(x, w12[...]) + b12[...], 0.0)
    x1_o[...] = _bn_full(x, g1[...], be1[...])


def _node_pre(data, *ws):
    return pl.pallas_call(
        _node_pre_body,
        out_shape=jax.ShapeDtypeStruct((N, D), _F32),
    )(data, *ws)


def _edge1_body(gs, gr, w1, b1, w2, b2, t2_o, st_o, acc):
    i = pl.program_id(0)
    x = jnp.concatenate([gs[...], gr[...]], axis=1)
    h = jnp.maximum(_dot(x, w1[...]) + b1[...], 0.0)
    t = jnp.maximum(_dot(h, w2[...]) + b2[...], 0.0)
    t2_o[...] = t

    @pl.when(i == 0)
    def _():
        acc[...] = jnp.zeros_like(acc)

    acc[0:1, :] += jnp.sum(t, 0, keepdims=True)
    acc[1:2, :] += jnp.sum(t * t, 0, keepdims=True)

    @pl.when(i == NB - 1)
    def _():
        st_o[...] = acc[...]


def _edge1(gs, gr, w1, b1, w2, b2):
    return pl.pallas_call(
        _edge1_body,
        grid=(NB,),
        in_specs=[
            pl.BlockSpec((BE, D), lambda i: (i, 0)),
            pl.BlockSpec((BE, D), lambda i: (i, 0)),
            pl.BlockSpec((2 * D, D), lambda i: (0, 0)),
            pl.BlockSpec((1, D), lambda i: (0, 0)),
            pl.BlockSpec((D, D), lambda i: (0, 0)),
            pl.BlockSpec((1, D), lambda i: (0, 0)),
        ],
        out_specs=[
            pl.BlockSpec((BE, D), lambda i: (i, 0)),
            pl.BlockSpec((8, D), lambda i: (0, 0)),
        ],
        out_shape=[
            jax.ShapeDtypeStruct((E, D), _F32),
            jax.ShapeDtypeStruct((8, D), _F32),
        ],
        scratch_shapes=[pltpu.VMEM((8, D), _F32)],
    )(gs, gr, w1, b1, w2, b2)


def _edge1y_body(t2, st, g2, be2, y_o):
    scale, shift = _scale_shift(st, g2[...], be2[...])
    y_o[...] = t2[...] * scale + shift


def _edge1y(t2, st, g2, be2):
    return pl.pallas_call(
        _edge1y_body,
        grid=(NB,),
        in_specs=[
            pl.BlockSpec((BE, D), lambda i: (i, 0)),
            pl.BlockSpec((8, D), lambda i: (0, 0)),
            pl.BlockSpec((1, D), lambda i: (0, 0)),
            pl.BlockSpec((1, D), lambda i: (0, 0)),
        ],
        out_specs=pl.BlockSpec((BE, D), lambda i: (i, 0)),
        out_shape=jax.ShapeDtypeStruct((E, D), _F32),
    )(t2, st, g2, be2)


def _node_mid_body(nsum, w31, b31, w32, b32, g3, be3, x3_o):
    n = nsum[...] * (1.0 / N)
    x = jnp.maximum(_dot(n, w31[...]) + b31[...], 0.0)
    x = jnp.maximum(_dot(x, w32[...]) + b32[...], 0.0)
    x3_o[...] = _bn_full(x, g3[...], be3[...])


def _node_mid(nsum, *ws):
    return pl.pallas_call(
        _node_mid_body,
        out_shape=jax.ShapeDtypeStruct((N, D), _F32),
    )(nsum, *ws)


def _edge2_body(gs, gr, y2, w1, b1, w2, b2, t4_o, st_o, acc):
    i = pl.program_id(0)
    x = jnp.concatenate([gs[...], gr[...], y2[...]], axis=1)
    h = jnp.maximum(_dot(x, w1[...]) + b1[...], 0.0)
    t = jnp.maximum(_dot(h, w2[...]) + b2[...], 0.0)
    t4_o[...] = t

    @pl.when(i == 0)
    def _():
        acc[...] = jnp.zeros_like(acc)

    acc[0:1, :] += jnp.sum(t, 0, keepdims=True)
    acc[1:2, :] += jnp.sum(t * t, 0, keepdims=True)

    @pl.when(i == NB - 1)
    def _():
        st_o[...] = acc[...]


def _edge2(gs, gr, y2, w1, b1, w2, b2):
    return pl.pallas_call(
        _edge2_body,
        grid=(NB,),
        in_specs=[
            pl.BlockSpec((BE, D), lambda i: (i, 0)),
            pl.BlockSpec((BE, D), lambda i: (i, 0)),
            pl.BlockSpec((BE, D), lambda i: (i, 0)),
            pl.BlockSpec((3 * D, D), lambda i: (0, 0)),
            pl.BlockSpec((1, D), lambda i: (0, 0)),
            pl.BlockSpec((D, D), lambda i: (0, 0)),
            pl.BlockSpec((1, D), lambda i: (0, 0)),
        ],
        out_specs=[
            pl.BlockSpec((BE, D), lambda i: (i, 0)),
            pl.BlockSpec((8, D), lambda i: (0, 0)),
        ],
        out_shape=[
            jax.ShapeDtypeStruct((E, D), _F32),
            jax.ShapeDtypeStruct((8, D), _F32),
        ],
        scratch_shapes=[pltpu.VMEM((8, D), _F32)],
    )(gs, gr, y2, w1, b1, w2, b2)


def _edge3_body(t4, st, g4, be4, wout, bout, gum, gds, gdr,
                wm01, bm01, wm02, bm02, wm11, bm11, wm12, bm12,
                ed_o, pr_o, am_o):
    scale, shift = _scale_shift(st, g4[...], be4[...])
    y4 = t4[...] * scale + shift
    lg = _dot(y4, wout[...]) + bout[...]

    u = (lg + gum[...]) / TAU
    u = u - jnp.max(u, axis=-1, keepdims=True)
    eu = jnp.exp(u)
    ed = eu / jnp.sum(eu, axis=-1, keepdims=True)
    ed_o[...] = ed

    v = lg - jnp.max(lg, axis=-1, keepdims=True)
    ev = jnp.exp(v)
    pr_o[...] = ev / jnp.sum(ev, axis=-1, keepdims=True)

    pm = jnp.concatenate([gds[...], gdr[...]], axis=1)
    m0 = jnp.maximum(_dot(pm, wm01[...]) + bm01[...], 0.0)
    m0 = jnp.maximum(_dot(m0, wm02[...]) + bm02[...], 0.0)
    m1 = jnp.maximum(_dot(pm, wm11[...]) + bm11[...], 0.0)
    m1 = jnp.maximum(_dot(m1, wm12[...]) + bm12[...], 0.0)
    am_o[...] = m0 * ed[:, 0:1] + m1 * ed[:, 1:2]


def _edge3(t4, st, g4, be4, wout, bout, gum, gds, gdr, *ws):
    return pl.pallas_call(
        _edge3_body,
        grid=(NB,),
        in_specs=[
            pl.BlockSpec((BE, D), lambda i: (i, 0)),
            pl.BlockSpec((8, D), lambda i: (0, 0)),
            pl.BlockSpec((1, D), lambda i: (0, 0)),
            pl.BlockSpec((1, D), lambda i: (0, 0)),
            pl.BlockSpec((D, T), lambda i: (0, 0)),
            pl.BlockSpec((1, T), lambda i: (0, 0)),
            pl.BlockSpec((BE, T), lambda i: (i, 0)),
            pl.BlockSpec((BE, D), lambda i: (i, 0)),
            pl.BlockSpec((BE, D), lambda i: (i, 0)),
            pl.BlockSpec((2 * D, D), lambda i: (0, 0)),
            pl.BlockSpec((1, D), lambda i: (0, 0)),
            pl.BlockSpec((D, D), lambda i: (0, 0)),
            pl.BlockSpec((1, D), lambda i: (0, 0)),
            pl.BlockSpec((2 * D, D), lambda i: (0, 0)),
            pl.BlockSpec((1, D), lambda i: (0, 0)),
            pl.BlockSpec((D, D), lambda i: (0, 0)),
            pl.BlockSpec((1, D), lambda i: (0, 0)),
        ],
        out_specs=[
            pl.BlockSpec((BE, T), lambda i: (i, 0)),
            pl.BlockSpec((BE, T), lambda i: (i, 0)),
            pl.BlockSpec((BE, D), lambda i: (i, 0)),
        ],
        out_shape=[
            jax.ShapeDtypeStruct((E, T), _F32),
            jax.ShapeDtypeStruct((E, T), _F32),
            jax.ShapeDtypeStruct((E, D), _F32),
        ],
    )(t4, st, g4, be4, wout, bout, gum, gds, gdr, *ws)


def _node_out_body(msum, wd1, bd1, wd2, bd2, wd3, bd3, o):
    agg = msum[...] * (1.0 / N)
    x = jnp.maximum(_dot(agg, wd1[...]) + bd1[...], 0.0)
    x = jnp.maximum(_dot(x, wd2[...]) + bd2[...], 0.0)
    o[...] = _dot(x, wd3[...]) + bd3[...]


def _node_out(msum, *ws):
    return pl.pallas_call(
        _node_out_body,
        out_shape=jax.ShapeDtypeStruct((N, D), _F32),
    )(msum, *ws)


_NC = 2
_NS = 16
_NW = _NC * _NS
_GCH = 256
_GKI = _GCH // 128
_GEW = E // _NW
_GIT = _GEW // _GCH


def _sc_gather_pair(tab, send2d, recv2d):
    @functools.partial(
        pl.kernel,
        out_type=[
            jax.ShapeDtypeStruct((E, D), _F32),
            jax.ShapeDtypeStruct((E, D), _F32),
        ],
        mesh=plsc.VectorSubcoreMesh(core_axis_name="c", subcore_axis_name="s"),
        scratch_types=[
            pltpu.VMEM((_GKI, 128), jnp.int32),
            pltpu.VMEM((_GKI, 128), jnp.int32),
            pltpu.VMEM((_GCH, D), _F32),
            pltpu.VMEM((_GCH, D), _F32),
            pltpu.SemaphoreType.DMA,
        ],
    )
    def k(tab_h, s_h, r_h, os_h, or_h, si_v, ri_v, rs_v, rr_v, sem):
        wid = lax.axis_index("s") * _NC + lax.axis_index("c")
        row0 = wid * (_GEW // 128)

        @pl.loop(0, _GIT)
        def _(it):
            rowbase = row0 + it * _GKI
            pltpu.sync_copy(s_h.at[pl.ds(rowbase, _GKI)], si_v)
            pltpu.sync_copy(r_h.at[pl.ds(rowbase, _GKI)], ri_v)
            cps = []
            for j in range(_GKI):
                cps.append(pltpu.async_copy(
                    tab_h.at[si_v.at[j]], rs_v.at[pl.ds(j * 128, 128)], sem))
                cps.append(pltpu.async_copy(
                    tab_h.at[ri_v.at[j]], rr_v.at[pl.ds(j * 128, 128)], sem))
            for c in cps:
                c.wait()
            base = rowbase * 128
            pltpu.sync_copy(rs_v, os_h.at[pl.ds(base, _GCH)])
            pltpu.sync_copy(rr_v, or_h.at[pl.ds(base, _GCH)])

    return k(tab, send2d, recv2d)


def kernel(data, send_idx, recv_idx, params, gumbel_noise):
    p = params
    r1 = lambda b: b.reshape(1, -1)
    send2d = send_idx.reshape(E // 128, 128)
    recv2d = recv_idx.reshape(E // 128, 128)

    x1 = _node_pre(
        data,
        p["enc_mlp1_fc1_W"], r1(p["enc_mlp1_fc1_b"]),
        p["enc_mlp1_fc2_W"], r1(p["enc_mlp1_fc2_b"]),
        r1(p["enc_mlp1_bn_gamma"]), r1(p["enc_mlp1_bn_beta"]),
    )

    gs1, gr1 = _sc_gather_pair(x1, send2d, recv2d)
    t2, st2 = _edge1(gs1, gr1,
                     p["enc_mlp2_fc1_W"], r1(p["enc_mlp2_fc1_b"]),
                     p["enc_mlp2_fc2_W"], r1(p["enc_mlp2_fc2_b"]))

    y2 = _edge1y(t2, st2, r1(p["enc_mlp2_bn_gamma"]), r1(p["enc_mlp2_bn_beta"]))

    nsum = jax.ops.segment_sum(y2, recv_idx, num_segments=N)

    x3 = _node_mid(
        nsum,
        p["enc_mlp3_fc1_W"], r1(p["enc_mlp3_fc1_b"]),
        p["enc_mlp3_fc2_W"], r1(p["enc_mlp3_fc2_b"]),
        r1(p["enc_mlp3_bn_gamma"]), r1(p["enc_mlp3_bn_beta"]),
    )

    gs3, gr3 = _sc_gather_pair(x3, send2d, recv2d)
    t4, st4 = _edge2(gs3, gr3, y2,
                     p["enc_mlp4_fc1_W"], r1(p["enc_mlp4_fc1_b"]),
                     p["enc_mlp4_fc2_W"], r1(p["enc_mlp4_fc2_b"]))

    gds, gdr = _sc_gather_pair(data, send2d, recv2d)
    ed, prob, am = _edge3(
        t4, st4,
        r1(p["enc_mlp4_bn_gamma"]), r1(p["enc_mlp4_bn_beta"]),
        p["enc_fc_out_W"], r1(p["enc_fc_out_b"]),
        gumbel_noise, gds, gdr,
        p["dec_msg_fc1_0_W"], r1(p["dec_msg_fc1_0_b"]),
        p["dec_msg_fc2_0_W"], r1(p["dec_msg_fc2_0_b"]),
        p["dec_msg_fc1_1_W"], r1(p["dec_msg_fc1_1_b"]),
        p["dec_msg_fc2_1_W"], r1(p["dec_msg_fc2_1_b"]),
    )

    msum = jax.ops.segment_sum(am, recv_idx, num_segments=N)
    output = _node_out(
        msum,
        p["dec_out_fc1_W"], r1(p["dec_out_fc1_b"]),
        p["dec_out_fc2_W"], r1(p["dec_out_fc2_b"]),
        p["dec_out_fc3_W"], r1(p["dec_out_fc3_b"]),
    )

    graphs = jnp.zeros((T, N, N), _F32)
    for k in range(T):
        graphs = graphs.at[k, send_idx, recv_idx].set(ed[:, k])

    return (graphs, output, prob)
